# baseline (speedup 1.0000x reference)
import functools

import jax
import jax.numpy as jnp
from jax import lax
from jax.experimental import pallas as pl
from jax.experimental.pallas import tpu as pltpu
from jax.experimental.pallas import tpu_sc as plsc

VOCAB = 1000000
DIM = 64
NB = 4096
NL = 200

NC = 2
NS = 16
NW = NC * NS
CHUNK = 128
LH = NL // 8
NBUF = 8
DIST = 4
NTBUF = 2


def _make_gather():
  mesh = plsc.VectorSubcoreMesh(core_axis_name="c", subcore_axis_name="s")

  @functools.partial(
      pl.kernel,
      mesh=mesh,
      out_type=jax.ShapeDtypeStruct((NL, 8, NW, 8 * CHUNK), jnp.float32),
      scratch_types=[
          pltpu.VMEM((LH, 8, CHUNK), jnp.int32),
          pltpu.VMEM((NBUF, CHUNK, DIM), jnp.float32),
          pltpu.VMEM((NTBUF, 8, 8 * CHUNK), jnp.float32),
      ] + [pltpu.SemaphoreType.DMA] * (NBUF + NTBUF),
      compiler_params=pltpu.CompilerParams(
          use_tc_tiling_on_sc=False, needs_layout_passes=False),
  )
  def k(idx_hbm, table_hbm, out_hbm, idx_v, emb_v, embt_v, *sems):
    gsem = sems[:NBUF]
    wsem = sems[NBUF:]
    bh = lax.axis_index("s") * NC + lax.axis_index("c")
    pltpu.sync_copy(idx_hbm.at[:, bh], idx_v)

    def gather(lh, ll, b):
      pltpu.async_copy(table_hbm.at[idx_v.at[lh, ll]], emb_v.at[b], gsem[b])

    def wait_gather(b):
      pltpu.make_async_copy(
          table_hbm.at[idx_v.at[0, 0]], emb_v.at[b], gsem[b]).wait()

    def put(l, c):
      pltpu.async_copy(embt_v.at[c], out_hbm.at[l, :, bh], wsem[c])

    def wait_put(c):
      pltpu.make_async_copy(
          embt_v.at[c], out_hbm.at[0, :, bh], wsem[c]).wait()

    tok = lax.iota(jnp.int32, 16)

    def transpose(b, c):
      def f_body(f, carry):
        dh = f // 8
        off = (f % 8) * CHUNK
        feat = jnp.full((16,), f, jnp.int32)
        for tb in range(8):
          vals = plsc.load_gather(emb_v.at[b], [tok + (tb * 16), feat])
          embt_v[c, dh, pl.ds(off + tb * 16, 16)] = vals
        return carry

      lax.fori_loop(0, DIM, f_body, 0)

    for ll in range(DIST):
      gather(0, ll, ll)

    def lh_body(lh, carry):
      for ll in range(NBUF):
        l = lh * 8 + ll
        c = ll % NTBUF
        wait_gather(ll)
        if ll < NTBUF:
          @pl.when(lh > 0)
          def _():
            wait_put(c)
        else:
          wait_put(c)
        transpose(ll, c)
        put(l, c)
        nll = (ll + DIST) % NBUF
        if ll < DIST:
          gather(lh, ll + DIST, nll)
        else:

          @pl.when(lh < LH - 1)
          def _():
            gather(lh + 1, nll, nll)

      return carry

    lax.fori_loop(0, LH, lh_body, 0)

    for c in range(NTBUF):
      wait_put(c)

  return k


_gather = _make_gather()


def kernel(input_ids, table):
  idx = (input_ids.astype(jnp.int32)
         .reshape(NW, CHUNK, LH, 8).transpose(2, 0, 3, 1))
  out5 = _gather(idx, table)
  return (out5.reshape(NL, 8, NW, 8, CHUNK)
          .transpose(2, 4, 0, 1, 3).reshape(NB, NL, DIM))
